# baseline (speedup 1.0000x reference)
import functools

import jax
import jax.numpy as jnp
from jax import lax
from jax.experimental import pallas as pl
from jax.experimental.pallas import tpu as pltpu
from jax.experimental.pallas import tpu_sc as plsc

N = 10000
E = 320000
B = 1024
D = 128
DH = 32

NS = 16
NP = 10240
RPT = NP // NS
EPT = E // NS
CH = 128
NCH = 160
EPP = NCH * CH


def _front_body(x_ref, wo_ref, wi_ref, po_ref, pi_ref):
    x = x_ref[...]
    po_ref[...] = jnp.dot(x, wo_ref[...], preferred_element_type=jnp.float32)
    pi_ref[...] = jnp.dot(x, wi_ref[...], preferred_element_type=jnp.float32)


def _front(x, wo, wi):
    return pl.pallas_call(
        _front_body,
        out_shape=[
            jax.ShapeDtypeStruct((2 * NP, DH), jnp.float32),
            jax.ShapeDtypeStruct((2 * NP, DH), jnp.float32),
        ],
    )(x, wo, wi)


CR = 160
NCR = RPT // CR
BLKJ = 16
NBUF = 8


def _sc_body(po_hbm, pi_hbm, src_hbm, dst_hbm, bo_hbm, bi_hbm,
             a2o_hbm, a2i_hbm, cd_hbm, cs_hbm, h1_hbm,
             acc_o, acc_i, cd_sh, cs_sh,
             sidx, didx, gb0, gb1, gb2, gb3, gb4, gb5, gb6, gb7,
             st_a, st_b, st_h,
             cd_l, cs_l, ivd, ivs, ones_v, bo_v, bi_v,
             sm0, sm1, sm2, sm3, sm4, sm5, sm6, sm7, sem_c):
    c = lax.axis_index("c")
    s = lax.axis_index("s")
    row0 = s * RPT
    rows = pl.ds(row0, RPT)
    f32 = jnp.float32

    def _chunk(k):
        return pl.ds(row0 + k * CR, CR)

    def _zero_rows(r, _):
        st_h[r, pl.ds(0, 16)] = jnp.zeros((16,), f32)
        st_h[r, pl.ds(16, 16)] = jnp.zeros((16,), f32)
        return 0

    lax.fori_loop(0, CR, _zero_rows, 0)

    def _zero_acc(k, _):
        pltpu.sync_copy(st_h, acc_o.at[_chunk(k), :])
        pltpu.sync_copy(st_h, acc_i.at[_chunk(k), :])
        return 0

    lax.fori_loop(0, NCR, _zero_acc, 0)

    def _zero_vec(k, _):
        ivd[pl.ds(k * 16, 16)] = jnp.zeros((16,), f32)
        return 0

    lax.fori_loop(0, RPT // 16, _zero_vec, 0)
    pltpu.sync_copy(ivd, cd_sh.at[rows])
    pltpu.sync_copy(ivd, cs_sh.at[rows])

    def _fill_ones(k, _):
        ones_v[pl.ds(k * 16, 16)] = jnp.zeros((16,), f32) + 1.0
        return 0

    lax.fori_loop(0, CH // 16, _fill_ones, 0)
    pltpu.sync_copy(bo_hbm, bo_v)
    pltpu.sync_copy(bi_hbm, bi_v)

    plsc.subcore_barrier()

    gbs = [gb0, gb1, gb2, gb3, gb4, gb5, gb6, gb7]
    sms = [sm0, sm1, sm2, sm3, sm4, sm5, sm6, sm7]

    def _dir(tab, gather_src, acc, with_counts):
        def blk_body(b, _):
            jsl = pl.ds(b * BLKJ, BLKJ)
            pltpu.sync_copy(src_hbm.at[c, s, jsl], sidx)
            pltpu.sync_copy(dst_hbm.at[c, s, jsl], didx)
            gref, sref = (sidx, didx) if gather_src else (didx, sidx)

            def batch(q, _q):
                j0 = q * NBUF
                for u in range(NBUF):
                    pltpu.async_copy(tab.at[gref.at[j0 + u]], gbs[u], sms[u])
                for u in range(NBUF):
                    pltpu.make_async_copy(
                        tab.at[gref.at[j0 + u]], gbs[u], sms[u]).wait()
                    pltpu.async_copy(gbs[u], acc.at[sref.at[j0 + u]], sms[u],
                                     add=True)
                    if with_counts:
                        pltpu.async_copy(ones_v, cd_sh.at[didx.at[j0 + u]],
                                         sem_c, add=True)
                        pltpu.async_copy(ones_v, cs_sh.at[sidx.at[j0 + u]],
                                         sem_c, add=True)
                for u in range(NBUF):
                    pltpu.make_async_copy(
                        gbs[u], acc.at[sref.at[j0 + u]], sms[u]).wait()
                if with_counts:
                    for u in range(NBUF):
                        pltpu.make_async_copy(
                            ones_v, cd_sh.at[didx.at[j0 + u]], sem_c).wait()
                        pltpu.make_async_copy(
                            ones_v, cs_sh.at[sidx.at[j0 + u]], sem_c).wait()
                return 0

            lax.fori_loop(0, BLKJ // NBUF, batch, 0)
            return 0

        lax.fori_loop(0, NCH // BLKJ, blk_body, 0)

    _dir(po_hbm.at[c], True, acc_o, True)
    _dir(pi_hbm.at[c], False, acc_i, False)
    plsc.subcore_barrier()

    pltpu.sync_copy(cd_sh.at[rows], cd_l)
    pltpu.sync_copy(cs_sh.at[rows], cs_l)

    def _inv(k, _):
        sl = pl.ds(k * 16, 16)
        one = jnp.zeros((16,), f32) + 1.0
        ivd[sl] = one / jnp.maximum(cd_l[sl], 1.0)
        ivs[sl] = one / jnp.maximum(cs_l[sl], 1.0)
        return 0

    lax.fori_loop(0, RPT // 16, _inv, 0)

    b1lo = bo_v[pl.ds(0, 16)] + bi_v[pl.ds(0, 16)]
    b1hi = bo_v[pl.ds(16, 16)] + bi_v[pl.ds(16, 16)]

    def _h1_chunk(k, _):
        pltpu.sync_copy(acc_o.at[_chunk(k), :], st_a)
        pltpu.sync_copy(acc_i.at[_chunk(k), :], st_b)

        def _h1_row(r, _r):
            i16 = jnp.zeros((16,), jnp.int32) + (k * CR + r)
            vd = plsc.load_gather(ivd, [i16])
            vs = plsc.load_gather(ivs, [i16])
            lo = st_a[r, pl.ds(0, 16)] * vd + st_b[r, pl.ds(0, 16)] * vs + b1lo
            hi = st_a[r, pl.ds(16, 16)] * vd + st_b[r, pl.ds(16, 16)] * vs + b1hi
            st_h[r, pl.ds(0, 16)] = jnp.maximum(lo, 0.0)
            st_h[r, pl.ds(16, 16)] = jnp.maximum(hi, 0.0)
            return 0

        lax.fori_loop(0, CR, _h1_row, 0)
        pltpu.sync_copy(st_h, h1_hbm.at[c, _chunk(k), :])
        return 0

    lax.fori_loop(0, NCR, _h1_chunk, 0)

    def _zero_rows2(r, _):
        st_a[r, pl.ds(0, 16)] = jnp.zeros((16,), f32)
        st_a[r, pl.ds(16, 16)] = jnp.zeros((16,), f32)
        return 0

    lax.fori_loop(0, CR, _zero_rows2, 0)

    def _zero_acc2(k, _):
        pltpu.sync_copy(st_a, acc_o.at[_chunk(k), :])
        pltpu.sync_copy(st_a, acc_i.at[_chunk(k), :])
        return 0

    lax.fori_loop(0, NCR, _zero_acc2, 0)
    plsc.subcore_barrier()

    _dir(h1_hbm.at[c], True, acc_o, False)
    _dir(h1_hbm.at[c], False, acc_i, False)
    plsc.subcore_barrier()

    def _out(k, _):
        pltpu.sync_copy(acc_o.at[_chunk(k), :], st_a)
        pltpu.sync_copy(st_a, a2o_hbm.at[c, _chunk(k), :])
        pltpu.sync_copy(acc_i.at[_chunk(k), :], st_b)
        pltpu.sync_copy(st_b, a2i_hbm.at[c, _chunk(k), :])
        return 0

    lax.fori_loop(0, NCR, _out, 0)
    pltpu.sync_copy(cd_l, cd_hbm.at[c, rows])
    pltpu.sync_copy(cs_l, cs_hbm.at[c, rows])


def _sc_aggregate(po, pi, src_t, dst_t, b_o1, b_i1):
    f32 = jnp.float32
    kern = pl.kernel(
        _sc_body,
        out_type=[
            jax.ShapeDtypeStruct((2, NP, DH), f32),
            jax.ShapeDtypeStruct((2, NP, DH), f32),
            jax.ShapeDtypeStruct((2, NP), f32),
            jax.ShapeDtypeStruct((2, NP), f32),
            jax.ShapeDtypeStruct((2, NP, DH), f32),
        ],
        mesh=plsc.VectorSubcoreMesh(core_axis_name="c", subcore_axis_name="s"),
        compiler_params=pltpu.CompilerParams(
            needs_layout_passes=False, use_tc_tiling_on_sc=False),
        scratch_types=[
            pltpu.VMEM_SHARED((NP, DH), f32),
            pltpu.VMEM_SHARED((NP, DH), f32),
            pltpu.VMEM_SHARED((NP,), f32),
            pltpu.VMEM_SHARED((NP,), f32),
            pltpu.VMEM((BLKJ, CH), jnp.int32),
            pltpu.VMEM((BLKJ, CH), jnp.int32),
        ] + [pltpu.VMEM((CH, DH), f32)] * 8 + [
            pltpu.VMEM((CR, DH), f32),
            pltpu.VMEM((CR, DH), f32),
            pltpu.VMEM((CR, DH), f32),
            pltpu.VMEM((RPT,), f32),
            pltpu.VMEM((RPT,), f32),
            pltpu.VMEM((RPT,), f32),
            pltpu.VMEM((RPT,), f32),
            pltpu.VMEM((CH,), f32),
            pltpu.VMEM((DH,), f32),
            pltpu.VMEM((DH,), f32),
        ] + [pltpu.SemaphoreType.DMA] * 9 + [
        ],
    )
    return kern(po, pi, src_t, dst_t, b_o1, b_i1)


BLKC = 1024
NB = NP // BLKC


def _finish_body(a2o_ref, a2i_ref, cd_ref, cs_ref, n2g_ref,
                 wo2_ref, bo2_ref, wi2_ref, bi2_ref,
                 rel_ref, wr_ref, br_ref, hot_ref, rr_ref,
                 pos_ref, neg_ref, gsum, gcnt, g0s):
    g = pl.program_id(0)
    nb = pl.program_id(1)

    @pl.when(nb == 0)
    def _init():
        gsum[...] = jnp.zeros_like(gsum)
        gcnt[...] = jnp.zeros_like(gcnt)

    cd = cd_ref[0, 0, :]
    cs = cs_ref[0, 0, :]
    mo = a2o_ref[0] * (1.0 / jnp.maximum(cd, 1.0))[:, None]
    mi = a2i_ref[0] * (1.0 / jnp.maximum(cs, 1.0))[:, None]
    h2 = jnp.dot(mo, wo2_ref[...], preferred_element_type=jnp.float32)
    h2 = h2 + jnp.dot(mi, wi2_ref[...], preferred_element_type=jnp.float32)
    h2 = jnp.maximum(h2 + bo2_ref[...] + bi2_ref[...], 0.0)

    n2 = n2g_ref[0, 0, :]
    oh = (n2[:, None] == lax.broadcasted_iota(jnp.int32, (BLKC, B), 1))
    oh = oh.astype(jnp.float32)
    gsum[...] += lax.dot_general(
        oh, h2, (((0,), (0,)), ((), ())),
        preferred_element_type=jnp.float32)
    gcnt[...] += jnp.sum(oh, axis=0)

    @pl.when((g == 0) & (nb == NB - 1))
    def _stash():
        g0s[...] = gsum[...] * (1.0 / jnp.maximum(gcnt[...], 1.0))[:, None]

    @pl.when((g == 1) & (nb == NB - 1))
    def _score():
        g1 = gsum[...] * (1.0 / jnp.maximum(gcnt[...], 1.0))[:, None]
        g0 = g0s[...]
        r = jnp.dot(rel_ref[...], wr_ref[...],
                    preferred_element_type=jnp.float32)
        r = jnp.maximum(r + br_ref[...], 0.0)
        ar = lax.broadcasted_iota(jnp.int32, (B, 1), 0)[:, 0]
        hot = hot_ref[...]
        rr = rr_ref[...]
        chidx = jnp.where(hot == 1, rr, ar)
        ctidx = jnp.where(hot == 0, rr, ar)
        iota_b = lax.broadcasted_iota(jnp.int32, (B, B), 1)
        hsel = (chidx[:, None] == iota_b).astype(jnp.float32)
        tsel = (ctidx[:, None] == iota_b).astype(jnp.float32)
        gh = jnp.dot(hsel, g0, preferred_element_type=jnp.float32)
        gt = jnp.dot(tsel, g1, preferred_element_type=jnp.float32)
        dpos = g0 + r - g1
        dneg = gh + r - gt
        pos_ref[...] = jnp.sqrt(jnp.sum(dpos * dpos, axis=1))
        neg_ref[...] = jnp.sqrt(jnp.sum(dneg * dneg, axis=1))


def _finish(a2o, a2i, cd, cs, n2g, wo2, bo2, wi2, bi2,
            rel, wr, br, hot, rr):
    f32 = jnp.float32
    full = lambda shape: pl.BlockSpec(shape, lambda g, nb: tuple(0 for _ in shape))
    return pl.pallas_call(
        _finish_body,
        grid=(2, NB),
        in_specs=[
            pl.BlockSpec((1, BLKC, DH), lambda g, nb: (g, nb, 0)),
            pl.BlockSpec((1, BLKC, DH), lambda g, nb: (g, nb, 0)),
            pl.BlockSpec((1, 1, BLKC), lambda g, nb: (g * NB + nb, 0, 0)),
            pl.BlockSpec((1, 1, BLKC), lambda g, nb: (g * NB + nb, 0, 0)),
            pl.BlockSpec((1, 1, BLKC), lambda g, nb: (g * NB + nb, 0, 0)),
            full((DH, D)),
            full((D,)),
            full((DH, D)),
            full((D,)),
            full((B, D)),
            full((D, D)),
            full((D,)),
            full((B,)),
            full((B,)),
        ],
        out_specs=[
            pl.BlockSpec((B,), lambda g, nb: (0,)),
            pl.BlockSpec((B,), lambda g, nb: (0,)),
        ],
        out_shape=[
            jax.ShapeDtypeStruct((B,), f32),
            jax.ShapeDtypeStruct((B,), f32),
        ],
        scratch_shapes=[
            pltpu.VMEM((B, D), f32),
            pltpu.VMEM((B,), f32),
            pltpu.VMEM((B, D), f32),
        ],
    )(a2o, a2i, cd, cs, n2g, wo2, bo2, wi2, bi2, rel, wr, br, hot, rr)


def kernel(subj_node_embs, rel_tok_embs, obj_node_embs, subj_edge_index,
           obj_edge_index, subj_node2graph, obj_node2graph, head_or_tail,
           random_rows, W_O1, b_O1, W_I1, b_I1, W_O2, b_O2, W_I2, b_I2,
           W_rel, b_rel):
    x = jnp.stack([subj_node_embs, obj_node_embs])
    x = jnp.pad(x, ((0, 0), (0, NP - N), (0, 0)))
    po, pi = _front(x.reshape(2 * NP, D), W_O1, W_I1)
    po = po.reshape(2, NP, DH)
    pi = pi.reshape(2, NP, DH)

    pad = N + (jnp.arange(EPP - EPT, dtype=jnp.int32) % (NP - N))
    pad = jnp.broadcast_to(pad, (2, NS, EPP - EPT))

    def _tiles(row):
        r = row.reshape(2, NS, EPT)
        return jnp.concatenate([r, pad], axis=2).reshape(2, NS, NCH, CH)

    src_t = _tiles(jnp.stack([subj_edge_index[0], obj_edge_index[0]]))
    dst_t = _tiles(jnp.stack([subj_edge_index[1], obj_edge_index[1]]))

    a2o, a2i, cd, cs, _h1 = _sc_aggregate(po, pi, src_t, dst_t, b_O1, b_I1)

    n2g = jnp.stack([subj_node2graph, obj_node2graph])
    n2g = jnp.pad(n2g, ((0, 0), (0, NP - N)), constant_values=B)

    cd = cd.reshape(2 * NB, 1, BLKC)
    cs = cs.reshape(2 * NB, 1, BLKC)
    n2g = n2g.reshape(2 * NB, 1, BLKC)

    pos, neg = _finish(a2o, a2i, cd, cs, n2g, W_O2, b_O2, W_I2, b_I2,
                       rel_tok_embs, W_rel, b_rel, head_or_tail, random_rows)
    return (pos, neg)


# speedup vs baseline: 1.1186x; 1.1186x over previous
import functools

import jax
import jax.numpy as jnp
from jax import lax
from jax.experimental import pallas as pl
from jax.experimental.pallas import tpu as pltpu
from jax.experimental.pallas import tpu_sc as plsc

N = 10000
E = 320000
B = 1024
D = 128
DH = 32

NS = 16
NP = 10240
RPT = NP // NS
EPT = E // NS
CH = 128
NCH = 160
EPP = NCH * CH


def _front_body(x_ref, wo_ref, wi_ref, po_ref, pi_ref):
    x = x_ref[...]
    po_ref[...] = jnp.dot(x, wo_ref[...], preferred_element_type=jnp.float32)
    pi_ref[...] = jnp.dot(x, wi_ref[...], preferred_element_type=jnp.float32)


def _front(x, wo, wi):
    return pl.pallas_call(
        _front_body,
        out_shape=[
            jax.ShapeDtypeStruct((2 * NP, DH), jnp.float32),
            jax.ShapeDtypeStruct((2 * NP, DH), jnp.float32),
        ],
    )(x, wo, wi)


CR = 160
NCR = RPT // CR
SCH = 512
NSC = NCH * CH // SCH
BLKQ = 8


def _sc_body(po_hbm, pi_hbm, src_hbm, dst_hbm, bo_hbm, bi_hbm,
             a2o_hbm, a2i_hbm, cd_hbm, cs_hbm, h1_hbm,
             acc_o, acc_i, cd_sh, cs_sh,
             sidx, didx, gb_a, gb_b,
             st_a, st_b, st_h,
             cd_l, cs_l, ivd, ivs, ones_v, bo_v, bi_v,
             sem_ga, sem_gb, sem_sa, sem_sb, sem_c):
    c = lax.axis_index("c")
    s = lax.axis_index("s")
    row0 = s * RPT
    rows = pl.ds(row0, RPT)
    f32 = jnp.float32

    def _chunk(k):
        return pl.ds(row0 + k * CR, CR)

    def _zero_rows(r, _):
        st_h[r, pl.ds(0, 16)] = jnp.zeros((16,), f32)
        st_h[r, pl.ds(16, 16)] = jnp.zeros((16,), f32)
        return 0

    lax.fori_loop(0, CR, _zero_rows, 0)

    def _zero_acc(k, _):
        pltpu.sync_copy(st_h, acc_o.at[_chunk(k), :])
        pltpu.sync_copy(st_h, acc_i.at[_chunk(k), :])
        return 0

    lax.fori_loop(0, NCR, _zero_acc, 0)

    def _zero_vec(k, _):
        ivd[pl.ds(k * 16, 16)] = jnp.zeros((16,), f32)
        return 0

    lax.fori_loop(0, RPT // 16, _zero_vec, 0)
    pltpu.sync_copy(ivd, cd_sh.at[rows])
    pltpu.sync_copy(ivd, cs_sh.at[rows])

    def _fill_ones(k, _):
        ones_v[pl.ds(k * 16, 16)] = jnp.zeros((16,), f32) + 1.0
        return 0

    lax.fori_loop(0, SCH // 16, _fill_ones, 0)
    pltpu.sync_copy(bo_hbm, bo_v)
    pltpu.sync_copy(bi_hbm, bi_v)

    plsc.subcore_barrier()

    def _dir(tab, gather_src, acc, with_counts):
        def blk_body(b, _):
            jsl = pl.ds(b * BLKQ * SCH, BLKQ * SCH)
            pltpu.sync_copy(src_hbm.at[c, s, jsl], sidx)
            pltpu.sync_copy(dst_hbm.at[c, s, jsl], didx)
            gref, sref = (sidx, didx) if gather_src else (didx, sidx)

            def _g(q):
                return gref.at[pl.ds(q * SCH, SCH)]

            def _s(q):
                return sref.at[pl.ds(q * SCH, SCH)]

            nsc = BLKQ
            pltpu.async_copy(tab.at[_g(0)], gb_a, sem_ga)

            def pair(jj, _p):
                q0 = 2 * jj
                q1 = q0 + 1
                if with_counts:
                    pltpu.async_copy(
                        ones_v, cd_sh.at[didx.at[pl.ds(q0 * SCH, SCH)]],
                        sem_c, add=True)
                    pltpu.async_copy(
                        ones_v, cs_sh.at[sidx.at[pl.ds(q0 * SCH, SCH)]],
                        sem_c, add=True)
                    pltpu.async_copy(
                        ones_v, cd_sh.at[didx.at[pl.ds(q1 * SCH, SCH)]],
                        sem_c, add=True)
                    pltpu.async_copy(
                        ones_v, cs_sh.at[sidx.at[pl.ds(q1 * SCH, SCH)]],
                        sem_c, add=True)
                pltpu.make_async_copy(tab.at[_g(q0)], gb_a, sem_ga).wait()
                d_gb = pltpu.async_copy(tab.at[_g(q1)], gb_b, sem_gb)
                d_sa = pltpu.async_copy(gb_a, acc.at[_s(q0)], sem_sa,
                                        add=True)
                d_gb.wait()
                d_sa.wait()

                @pl.when(jj < nsc // 2 - 1)
                def _next():
                    pltpu.async_copy(tab.at[_g(q0 + 2)], gb_a, sem_ga)

                d_sb = pltpu.async_copy(gb_b, acc.at[_s(q1)], sem_sb,
                                        add=True)
                d_sb.wait()
                if with_counts:
                    for qx in (q0, q1):
                        pltpu.make_async_copy(
                            ones_v, cd_sh.at[didx.at[pl.ds(qx * SCH, SCH)]],
                            sem_c).wait()
                        pltpu.make_async_copy(
                            ones_v, cs_sh.at[sidx.at[pl.ds(qx * SCH, SCH)]],
                            sem_c).wait()
                return 0

            lax.fori_loop(0, nsc // 2, pair, 0)
            return 0

        lax.fori_loop(0, NSC // BLKQ, blk_body, 0)

    _dir(po_hbm.at[c], True, acc_o, True)
    _dir(pi_hbm.at[c], False, acc_i, False)
    plsc.subcore_barrier()

    pltpu.sync_copy(cd_sh.at[rows], cd_l)
    pltpu.sync_copy(cs_sh.at[rows], cs_l)

    def _inv(k, _):
        sl = pl.ds(k * 16, 16)
        one = jnp.zeros((16,), f32) + 1.0
        ivd[sl] = one / jnp.maximum(cd_l[sl], 1.0)
        ivs[sl] = one / jnp.maximum(cs_l[sl], 1.0)
        return 0

    lax.fori_loop(0, RPT // 16, _inv, 0)

    b1lo = bo_v[pl.ds(0, 16)] + bi_v[pl.ds(0, 16)]
    b1hi = bo_v[pl.ds(16, 16)] + bi_v[pl.ds(16, 16)]

    def _h1_chunk(k, _):
        pltpu.sync_copy(acc_o.at[_chunk(k), :], st_a)
        pltpu.sync_copy(acc_i.at[_chunk(k), :], st_b)

        def _h1_row(r, _r):
            i16 = jnp.zeros((16,), jnp.int32) + (k * CR + r)
            vd = plsc.load_gather(ivd, [i16])
            vs = plsc.load_gather(ivs, [i16])
            lo = st_a[r, pl.ds(0, 16)] * vd + st_b[r, pl.ds(0, 16)] * vs + b1lo
            hi = st_a[r, pl.ds(16, 16)] * vd + st_b[r, pl.ds(16, 16)] * vs + b1hi
            st_h[r, pl.ds(0, 16)] = jnp.maximum(lo, 0.0)
            st_h[r, pl.ds(16, 16)] = jnp.maximum(hi, 0.0)
            return 0

        lax.fori_loop(0, CR, _h1_row, 0)
        pltpu.sync_copy(st_h, h1_hbm.at[c, _chunk(k), :])
        return 0

    lax.fori_loop(0, NCR, _h1_chunk, 0)

    def _zero_rows2(r, _):
        st_a[r, pl.ds(0, 16)] = jnp.zeros((16,), f32)
        st_a[r, pl.ds(16, 16)] = jnp.zeros((16,), f32)
        return 0

    lax.fori_loop(0, CR, _zero_rows2, 0)

    def _zero_acc2(k, _):
        pltpu.sync_copy(st_a, acc_o.at[_chunk(k), :])
        pltpu.sync_copy(st_a, acc_i.at[_chunk(k), :])
        return 0

    lax.fori_loop(0, NCR, _zero_acc2, 0)
    plsc.subcore_barrier()

    _dir(h1_hbm.at[c], True, acc_o, False)
    _dir(h1_hbm.at[c], False, acc_i, False)
    plsc.subcore_barrier()

    def _out(k, _):
        pltpu.sync_copy(acc_o.at[_chunk(k), :], st_a)
        pltpu.sync_copy(st_a, a2o_hbm.at[c, _chunk(k), :])
        pltpu.sync_copy(acc_i.at[_chunk(k), :], st_b)
        pltpu.sync_copy(st_b, a2i_hbm.at[c, _chunk(k), :])
        return 0

    lax.fori_loop(0, NCR, _out, 0)
    pltpu.sync_copy(cd_l, cd_hbm.at[c, rows])
    pltpu.sync_copy(cs_l, cs_hbm.at[c, rows])


def _sc_aggregate(po, pi, src_t, dst_t, b_o1, b_i1):
    f32 = jnp.float32
    kern = pl.kernel(
        _sc_body,
        out_type=[
            jax.ShapeDtypeStruct((2, NP, DH), f32),
            jax.ShapeDtypeStruct((2, NP, DH), f32),
            jax.ShapeDtypeStruct((2, NP), f32),
            jax.ShapeDtypeStruct((2, NP), f32),
            jax.ShapeDtypeStruct((2, NP, DH), f32),
        ],
        mesh=plsc.VectorSubcoreMesh(core_axis_name="c", subcore_axis_name="s"),
        compiler_params=pltpu.CompilerParams(
            needs_layout_passes=False, use_tc_tiling_on_sc=False),
        scratch_types=[
            pltpu.VMEM_SHARED((NP, DH), f32),
            pltpu.VMEM_SHARED((NP, DH), f32),
            pltpu.VMEM_SHARED((NP,), f32),
            pltpu.VMEM_SHARED((NP,), f32),
            pltpu.VMEM((BLKQ * SCH,), jnp.int32),
            pltpu.VMEM((BLKQ * SCH,), jnp.int32),
            pltpu.VMEM((SCH, DH), f32),
            pltpu.VMEM((SCH, DH), f32),
            pltpu.VMEM((CR, DH), f32),
            pltpu.VMEM((CR, DH), f32),
            pltpu.VMEM((CR, DH), f32),
            pltpu.VMEM((RPT,), f32),
            pltpu.VMEM((RPT,), f32),
            pltpu.VMEM((RPT,), f32),
            pltpu.VMEM((RPT,), f32),
            pltpu.VMEM((SCH,), f32),
            pltpu.VMEM((DH,), f32),
            pltpu.VMEM((DH,), f32),
        ] + [pltpu.SemaphoreType.DMA] * 5 + [
        ],
    )
    return kern(po, pi, src_t, dst_t, b_o1, b_i1)


BLKC = 1024
NB = NP // BLKC


def _finish_body(a2o_ref, a2i_ref, cd_ref, cs_ref, n2g_ref,
                 wo2_ref, bo2_ref, wi2_ref, bi2_ref,
                 rel_ref, wr_ref, br_ref, hot_ref, rr_ref,
                 pos_ref, neg_ref, gsum, gcnt, g0s):
    g = pl.program_id(0)
    nb = pl.program_id(1)

    @pl.when(nb == 0)
    def _init():
        gsum[...] = jnp.zeros_like(gsum)
        gcnt[...] = jnp.zeros_like(gcnt)

    cd = cd_ref[0, 0, :]
    cs = cs_ref[0, 0, :]
    mo = a2o_ref[0] * (1.0 / jnp.maximum(cd, 1.0))[:, None]
    mi = a2i_ref[0] * (1.0 / jnp.maximum(cs, 1.0))[:, None]
    h2 = jnp.dot(mo, wo2_ref[...], preferred_element_type=jnp.float32)
    h2 = h2 + jnp.dot(mi, wi2_ref[...], preferred_element_type=jnp.float32)
    h2 = jnp.maximum(h2 + bo2_ref[...] + bi2_ref[...], 0.0)

    n2 = n2g_ref[0, 0, :]
    oh = (n2[:, None] == lax.broadcasted_iota(jnp.int32, (BLKC, B), 1))
    oh = oh.astype(jnp.float32)
    gsum[...] += lax.dot_general(
        oh, h2, (((0,), (0,)), ((), ())),
        preferred_element_type=jnp.float32)
    gcnt[...] += jnp.sum(oh, axis=0)

    @pl.when((g == 0) & (nb == NB - 1))
    def _stash():
        g0s[...] = gsum[...] * (1.0 / jnp.maximum(gcnt[...], 1.0))[:, None]

    @pl.when((g == 1) & (nb == NB - 1))
    def _score():
        g1 = gsum[...] * (1.0 / jnp.maximum(gcnt[...], 1.0))[:, None]
        g0 = g0s[...]
        r = jnp.dot(rel_ref[...], wr_ref[...],
                    preferred_element_type=jnp.float32)
        r = jnp.maximum(r + br_ref[...], 0.0)
        ar = lax.broadcasted_iota(jnp.int32, (B, 1), 0)[:, 0]
        hot = hot_ref[...]
        rr = rr_ref[...]
        chidx = jnp.where(hot == 1, rr, ar)
        ctidx = jnp.where(hot == 0, rr, ar)
        iota_b = lax.broadcasted_iota(jnp.int32, (B, B), 1)
        hsel = (chidx[:, None] == iota_b).astype(jnp.float32)
        tsel = (ctidx[:, None] == iota_b).astype(jnp.float32)
        gh = jnp.dot(hsel, g0, preferred_element_type=jnp.float32)
        gt = jnp.dot(tsel, g1, preferred_element_type=jnp.float32)
        dpos = g0 + r - g1
        dneg = gh + r - gt
        pos_ref[...] = jnp.sqrt(jnp.sum(dpos * dpos, axis=1))
        neg_ref[...] = jnp.sqrt(jnp.sum(dneg * dneg, axis=1))


def _finish(a2o, a2i, cd, cs, n2g, wo2, bo2, wi2, bi2,
            rel, wr, br, hot, rr):
    f32 = jnp.float32
    full = lambda shape: pl.BlockSpec(shape, lambda g, nb: tuple(0 for _ in shape))
    return pl.pallas_call(
        _finish_body,
        grid=(2, NB),
        in_specs=[
            pl.BlockSpec((1, BLKC, DH), lambda g, nb: (g, nb, 0)),
            pl.BlockSpec((1, BLKC, DH), lambda g, nb: (g, nb, 0)),
            pl.BlockSpec((1, 1, BLKC), lambda g, nb: (g * NB + nb, 0, 0)),
            pl.BlockSpec((1, 1, BLKC), lambda g, nb: (g * NB + nb, 0, 0)),
            pl.BlockSpec((1, 1, BLKC), lambda g, nb: (g * NB + nb, 0, 0)),
            full((DH, D)),
            full((D,)),
            full((DH, D)),
            full((D,)),
            full((B, D)),
            full((D, D)),
            full((D,)),
            full((B,)),
            full((B,)),
        ],
        out_specs=[
            pl.BlockSpec((B,), lambda g, nb: (0,)),
            pl.BlockSpec((B,), lambda g, nb: (0,)),
        ],
        out_shape=[
            jax.ShapeDtypeStruct((B,), f32),
            jax.ShapeDtypeStruct((B,), f32),
        ],
        scratch_shapes=[
            pltpu.VMEM((B, D), f32),
            pltpu.VMEM((B,), f32),
            pltpu.VMEM((B, D), f32),
        ],
    )(a2o, a2i, cd, cs, n2g, wo2, bo2, wi2, bi2, rel, wr, br, hot, rr)


def kernel(subj_node_embs, rel_tok_embs, obj_node_embs, subj_edge_index,
           obj_edge_index, subj_node2graph, obj_node2graph, head_or_tail,
           random_rows, W_O1, b_O1, W_I1, b_I1, W_O2, b_O2, W_I2, b_I2,
           W_rel, b_rel):
    x = jnp.stack([subj_node_embs, obj_node_embs])
    x = jnp.pad(x, ((0, 0), (0, NP - N), (0, 0)))
    po, pi = _front(x.reshape(2 * NP, D), W_O1, W_I1)
    po = po.reshape(2, NP, DH)
    pi = pi.reshape(2, NP, DH)

    pad = N + (jnp.arange(EPP - EPT, dtype=jnp.int32) % (NP - N))
    pad = jnp.broadcast_to(pad, (2, NS, EPP - EPT))

    def _tiles(row):
        r = row.reshape(2, NS, EPT)
        return jnp.concatenate([r, pad], axis=2).reshape(2, NS, NSC * SCH)

    src_t = _tiles(jnp.stack([subj_edge_index[0], obj_edge_index[0]]))
    dst_t = _tiles(jnp.stack([subj_edge_index[1], obj_edge_index[1]]))

    a2o, a2i, cd, cs, _h1 = _sc_aggregate(po, pi, src_t, dst_t, b_O1, b_I1)

    n2g = jnp.stack([subj_node2graph, obj_node2graph])
    n2g = jnp.pad(n2g, ((0, 0), (0, NP - N)), constant_values=B)

    cd = cd.reshape(2 * NB, 1, BLKC)
    cs = cs.reshape(2 * NB, 1, BLKC)
    n2g = n2g.reshape(2 * NB, 1, BLKC)

    pos, neg = _finish(a2o, a2i, cd, cs, n2g, W_O2, b_O2, W_I2, b_I2,
                       rel_tok_embs, W_rel, b_rel, head_or_tail, random_rows)
    return (pos, neg)


# speedup vs baseline: 1.1514x; 1.0294x over previous
import functools

import jax
import jax.numpy as jnp
from jax import lax
from jax.experimental import pallas as pl
from jax.experimental.pallas import tpu as pltpu
from jax.experimental.pallas import tpu_sc as plsc

N = 10000
E = 320000
B = 1024
D = 128
DH = 32

NS = 16
NP = 10240
RPT = NP // NS
EPT = E // NS
CH = 128
NCH = 160
EPP = NCH * CH


def _front_body(xs_ref, xo_ref, wo_ref, wi_ref, po_ref, pi_ref):
    wo = wo_ref[...]
    wi = wi_ref[...]
    zpad = jnp.zeros((NP - N, DH), jnp.float32)
    for g, xr in ((0, xs_ref), (1, xo_ref)):
        x = xr[...]
        po_ref[g, :N, :] = jnp.dot(x, wo, preferred_element_type=jnp.float32)
        po_ref[g, N:, :] = zpad
        pi_ref[g, :N, :] = jnp.dot(x, wi, preferred_element_type=jnp.float32)
        pi_ref[g, N:, :] = zpad


def _front(xs, xo, wo, wi):
    return pl.pallas_call(
        _front_body,
        out_shape=[
            jax.ShapeDtypeStruct((2, NP, DH), jnp.float32),
            jax.ShapeDtypeStruct((2, NP, DH), jnp.float32),
        ],
    )(xs, xo, wo, wi)


CR = 160
NCR = RPT // CR
SCH = 512
NSC = NCH * CH // SCH
BLKQ = 8


def _sc_body(po_hbm, pi_hbm, src_hbm, dst_hbm, bo_hbm, bi_hbm,
             a2o_hbm, a2i_hbm, cd_hbm, cs_hbm, h1_hbm,
             acc_o, acc_i, cd_sh, cs_sh,
             sidx, didx, gb_a, gb_b,
             st_a, st_b, st_h,
             cd_l, cs_l, ivd, ivs, ones_v, bo_v, bi_v,
             sem_ga, sem_gb, sem_sa, sem_sb, sem_c):
    c = lax.axis_index("c")
    s = lax.axis_index("s")
    row0 = s * RPT
    rows = pl.ds(row0, RPT)
    f32 = jnp.float32

    def _chunk(k):
        return pl.ds(row0 + k * CR, CR)

    def _zero_rows(r, _):
        st_h[r, pl.ds(0, 16)] = jnp.zeros((16,), f32)
        st_h[r, pl.ds(16, 16)] = jnp.zeros((16,), f32)
        return 0

    lax.fori_loop(0, CR, _zero_rows, 0)

    def _zero_acc(k, _):
        pltpu.sync_copy(st_h, acc_o.at[_chunk(k), :])
        pltpu.sync_copy(st_h, acc_i.at[_chunk(k), :])
        return 0

    lax.fori_loop(0, NCR, _zero_acc, 0)

    def _zero_vec(k, _):
        ivd[pl.ds(k * 16, 16)] = jnp.zeros((16,), f32)
        return 0

    lax.fori_loop(0, RPT // 16, _zero_vec, 0)
    pltpu.sync_copy(ivd, cd_sh.at[rows])
    pltpu.sync_copy(ivd, cs_sh.at[rows])

    def _fill_ones(k, _):
        ones_v[pl.ds(k * 16, 16)] = jnp.zeros((16,), f32) + 1.0
        return 0

    lax.fori_loop(0, SCH // 16, _fill_ones, 0)
    pltpu.sync_copy(bo_hbm, bo_v)
    pltpu.sync_copy(bi_hbm, bi_v)

    plsc.subcore_barrier()

    def _dir(tab, gather_src, acc, with_counts):
        def blk_body(b, _):
            jsl = pl.ds(b * BLKQ * SCH, BLKQ * SCH)
            pltpu.sync_copy(src_hbm.at[c, s, jsl], sidx)
            pltpu.sync_copy(dst_hbm.at[c, s, jsl], didx)
            gref, sref = (sidx, didx) if gather_src else (didx, sidx)

            def _g(q):
                return gref.at[pl.ds(q * SCH, SCH)]

            def _s(q):
                return sref.at[pl.ds(q * SCH, SCH)]

            nsc = BLKQ
            pltpu.async_copy(tab.at[_g(0)], gb_a, sem_ga)

            def pair(jj, _p):
                q0 = 2 * jj
                q1 = q0 + 1
                if with_counts:
                    pltpu.async_copy(
                        ones_v, cd_sh.at[didx.at[pl.ds(q0 * SCH, SCH)]],
                        sem_c, add=True)
                    pltpu.async_copy(
                        ones_v, cs_sh.at[sidx.at[pl.ds(q0 * SCH, SCH)]],
                        sem_c, add=True)
                    pltpu.async_copy(
                        ones_v, cd_sh.at[didx.at[pl.ds(q1 * SCH, SCH)]],
                        sem_c, add=True)
                    pltpu.async_copy(
                        ones_v, cs_sh.at[sidx.at[pl.ds(q1 * SCH, SCH)]],
                        sem_c, add=True)
                pltpu.make_async_copy(tab.at[_g(q0)], gb_a, sem_ga).wait()
                d_gb = pltpu.async_copy(tab.at[_g(q1)], gb_b, sem_gb)
                d_sa = pltpu.async_copy(gb_a, acc.at[_s(q0)], sem_sa,
                                        add=True)
                d_gb.wait()
                d_sa.wait()

                @pl.when(jj < nsc // 2 - 1)
                def _next():
                    pltpu.async_copy(tab.at[_g(q0 + 2)], gb_a, sem_ga)

                d_sb = pltpu.async_copy(gb_b, acc.at[_s(q1)], sem_sb,
                                        add=True)
                d_sb.wait()
                if with_counts:
                    for qx in (q0, q1):
                        pltpu.make_async_copy(
                            ones_v, cd_sh.at[didx.at[pl.ds(qx * SCH, SCH)]],
                            sem_c).wait()
                        pltpu.make_async_copy(
                            ones_v, cs_sh.at[sidx.at[pl.ds(qx * SCH, SCH)]],
                            sem_c).wait()
                return 0

            lax.fori_loop(0, nsc // 2, pair, 0)
            return 0

        lax.fori_loop(0, NSC // BLKQ, blk_body, 0)

    _dir(po_hbm.at[c], True, acc_o, True)
    _dir(pi_hbm.at[c], False, acc_i, False)
    plsc.subcore_barrier()

    pltpu.sync_copy(cd_sh.at[rows], cd_l)
    pltpu.sync_copy(cs_sh.at[rows], cs_l)

    def _inv(k, _):
        sl = pl.ds(k * 16, 16)
        one = jnp.zeros((16,), f32) + 1.0
        ivd[sl] = one / jnp.maximum(cd_l[sl], 1.0)
        ivs[sl] = one / jnp.maximum(cs_l[sl], 1.0)
        return 0

    lax.fori_loop(0, RPT // 16, _inv, 0)

    b1lo = bo_v[pl.ds(0, 16)] + bi_v[pl.ds(0, 16)]
    b1hi = bo_v[pl.ds(16, 16)] + bi_v[pl.ds(16, 16)]

    def _h1_chunk(k, _):
        pltpu.sync_copy(acc_o.at[_chunk(k), :], st_a)
        pltpu.sync_copy(acc_i.at[_chunk(k), :], st_b)

        def _h1_row(r, _r):
            i16 = jnp.zeros((16,), jnp.int32) + (k * CR + r)
            vd = plsc.load_gather(ivd, [i16])
            vs = plsc.load_gather(ivs, [i16])
            lo = st_a[r, pl.ds(0, 16)] * vd + st_b[r, pl.ds(0, 16)] * vs + b1lo
            hi = st_a[r, pl.ds(16, 16)] * vd + st_b[r, pl.ds(16, 16)] * vs + b1hi
            st_h[r, pl.ds(0, 16)] = jnp.maximum(lo, 0.0)
            st_h[r, pl.ds(16, 16)] = jnp.maximum(hi, 0.0)
            return 0

        lax.fori_loop(0, CR, _h1_row, 0)
        pltpu.sync_copy(st_h, h1_hbm.at[c, _chunk(k), :])
        return 0

    lax.fori_loop(0, NCR, _h1_chunk, 0)

    def _zero_rows2(r, _):
        st_a[r, pl.ds(0, 16)] = jnp.zeros((16,), f32)
        st_a[r, pl.ds(16, 16)] = jnp.zeros((16,), f32)
        return 0

    lax.fori_loop(0, CR, _zero_rows2, 0)

    def _zero_acc2(k, _):
        pltpu.sync_copy(st_a, acc_o.at[_chunk(k), :])
        pltpu.sync_copy(st_a, acc_i.at[_chunk(k), :])
        return 0

    lax.fori_loop(0, NCR, _zero_acc2, 0)
    plsc.subcore_barrier()

    _dir(h1_hbm.at[c], True, acc_o, False)
    _dir(h1_hbm.at[c], False, acc_i, False)
    plsc.subcore_barrier()

    def _out(k, _):
        pltpu.sync_copy(acc_o.at[_chunk(k), :], st_a)
        pltpu.sync_copy(st_a, a2o_hbm.at[c, _chunk(k), :])
        pltpu.sync_copy(acc_i.at[_chunk(k), :], st_b)
        pltpu.sync_copy(st_b, a2i_hbm.at[c, _chunk(k), :])
        return 0

    lax.fori_loop(0, NCR, _out, 0)
    pltpu.sync_copy(cd_l, cd_hbm.at[c, rows])
    pltpu.sync_copy(cs_l, cs_hbm.at[c, rows])


def _sc_aggregate(po, pi, src_t, dst_t, b_o1, b_i1):
    f32 = jnp.float32
    kern = pl.kernel(
        _sc_body,
        out_type=[
            jax.ShapeDtypeStruct((2, NP, DH), f32),
            jax.ShapeDtypeStruct((2, NP, DH), f32),
            jax.ShapeDtypeStruct((2, NP), f32),
            jax.ShapeDtypeStruct((2, NP), f32),
            jax.ShapeDtypeStruct((2, NP, DH), f32),
        ],
        mesh=plsc.VectorSubcoreMesh(core_axis_name="c", subcore_axis_name="s"),
        compiler_params=pltpu.CompilerParams(
            needs_layout_passes=False, use_tc_tiling_on_sc=False),
        scratch_types=[
            pltpu.VMEM_SHARED((NP, DH), f32),
            pltpu.VMEM_SHARED((NP, DH), f32),
            pltpu.VMEM_SHARED((NP,), f32),
            pltpu.VMEM_SHARED((NP,), f32),
            pltpu.VMEM((BLKQ * SCH,), jnp.int32),
            pltpu.VMEM((BLKQ * SCH,), jnp.int32),
            pltpu.VMEM((SCH, DH), f32),
            pltpu.VMEM((SCH, DH), f32),
            pltpu.VMEM((CR, DH), f32),
            pltpu.VMEM((CR, DH), f32),
            pltpu.VMEM((CR, DH), f32),
            pltpu.VMEM((RPT,), f32),
            pltpu.VMEM((RPT,), f32),
            pltpu.VMEM((RPT,), f32),
            pltpu.VMEM((RPT,), f32),
            pltpu.VMEM((SCH,), f32),
            pltpu.VMEM((DH,), f32),
            pltpu.VMEM((DH,), f32),
        ] + [pltpu.SemaphoreType.DMA] * 5 + [
        ],
    )
    return kern(po, pi, src_t, dst_t, b_o1, b_i1)


BLKC = 2048
NB = NP // BLKC


def _finish_body(a2o_ref, a2i_ref, cd_ref, cs_ref, n2g_ref,
                 wo2_ref, bo2_ref, wi2_ref, bi2_ref,
                 rel_ref, wr_ref, br_ref, hot_ref, rr_ref,
                 pos_ref, neg_ref, gsum, gcnt, g0s):
    g = pl.program_id(0)
    nb = pl.program_id(1)

    @pl.when(nb == 0)
    def _init():
        gsum[...] = jnp.zeros_like(gsum)
        gcnt[...] = jnp.zeros_like(gcnt)

    cd = cd_ref[0, 0, :]
    cs = cs_ref[0, 0, :]
    mo = a2o_ref[0] * (1.0 / jnp.maximum(cd, 1.0))[:, None]
    mi = a2i_ref[0] * (1.0 / jnp.maximum(cs, 1.0))[:, None]
    h2 = jnp.dot(mo, wo2_ref[...], preferred_element_type=jnp.float32)
    h2 = h2 + jnp.dot(mi, wi2_ref[...], preferred_element_type=jnp.float32)
    h2 = jnp.maximum(h2 + bo2_ref[...] + bi2_ref[...], 0.0)

    n2 = n2g_ref[0, 0, :]
    oh = (n2[:, None] == lax.broadcasted_iota(jnp.int32, (BLKC, B), 1))
    oh = oh.astype(jnp.float32)
    gsum[...] += lax.dot_general(
        oh, h2, (((0,), (0,)), ((), ())),
        preferred_element_type=jnp.float32)
    gcnt[...] += jnp.sum(oh, axis=0)

    @pl.when((g == 0) & (nb == NB - 1))
    def _stash():
        g0s[...] = gsum[...] * (1.0 / jnp.maximum(gcnt[...], 1.0))[:, None]

    @pl.when((g == 1) & (nb == NB - 1))
    def _score():
        g1 = gsum[...] * (1.0 / jnp.maximum(gcnt[...], 1.0))[:, None]
        g0 = g0s[...]
        r = jnp.dot(rel_ref[...], wr_ref[...],
                    preferred_element_type=jnp.float32)
        r = jnp.maximum(r + br_ref[...], 0.0)
        ar = lax.broadcasted_iota(jnp.int32, (B, 1), 0)[:, 0]
        hot = hot_ref[...]
        rr = rr_ref[...]
        chidx = jnp.where(hot == 1, rr, ar)
        ctidx = jnp.where(hot == 0, rr, ar)
        iota_b = lax.broadcasted_iota(jnp.int32, (B, B), 1)
        hsel = (chidx[:, None] == iota_b).astype(jnp.float32)
        tsel = (ctidx[:, None] == iota_b).astype(jnp.float32)
        gh = jnp.dot(hsel, g0, preferred_element_type=jnp.float32)
        gt = jnp.dot(tsel, g1, preferred_element_type=jnp.float32)
        dpos = g0 + r - g1
        dneg = gh + r - gt
        pos_ref[...] = jnp.sqrt(jnp.sum(dpos * dpos, axis=1))
        neg_ref[...] = jnp.sqrt(jnp.sum(dneg * dneg, axis=1))


def _finish(a2o, a2i, cd, cs, n2g, wo2, bo2, wi2, bi2,
            rel, wr, br, hot, rr):
    f32 = jnp.float32
    full = lambda shape: pl.BlockSpec(shape, lambda g, nb: tuple(0 for _ in shape))
    return pl.pallas_call(
        _finish_body,
        grid=(2, NB),
        in_specs=[
            pl.BlockSpec((1, BLKC, DH), lambda g, nb: (g, nb, 0)),
            pl.BlockSpec((1, BLKC, DH), lambda g, nb: (g, nb, 0)),
            pl.BlockSpec((1, 1, BLKC), lambda g, nb: (g * NB + nb, 0, 0)),
            pl.BlockSpec((1, 1, BLKC), lambda g, nb: (g * NB + nb, 0, 0)),
            pl.BlockSpec((1, 1, BLKC), lambda g, nb: (g * NB + nb, 0, 0)),
            full((DH, D)),
            full((D,)),
            full((DH, D)),
            full((D,)),
            full((B, D)),
            full((D, D)),
            full((D,)),
            full((B,)),
            full((B,)),
        ],
        out_specs=[
            pl.BlockSpec((B,), lambda g, nb: (0,)),
            pl.BlockSpec((B,), lambda g, nb: (0,)),
        ],
        out_shape=[
            jax.ShapeDtypeStruct((B,), f32),
            jax.ShapeDtypeStruct((B,), f32),
        ],
        scratch_shapes=[
            pltpu.VMEM((B, D), f32),
            pltpu.VMEM((B,), f32),
            pltpu.VMEM((B, D), f32),
        ],
    )(a2o, a2i, cd, cs, n2g, wo2, bo2, wi2, bi2, rel, wr, br, hot, rr)


def kernel(subj_node_embs, rel_tok_embs, obj_node_embs, subj_edge_index,
           obj_edge_index, subj_node2graph, obj_node2graph, head_or_tail,
           random_rows, W_O1, b_O1, W_I1, b_I1, W_O2, b_O2, W_I2, b_I2,
           W_rel, b_rel):
    po, pi = _front(subj_node_embs, obj_node_embs, W_O1, W_I1)

    pad = N + (jnp.arange(EPP - EPT, dtype=jnp.int32) % (NP - N))
    pad = jnp.broadcast_to(pad, (2, NS, EPP - EPT))

    def _tiles(row):
        r = row.reshape(2, NS, EPT)
        return jnp.concatenate([r, pad], axis=2).reshape(2, NS, NSC * SCH)

    src_t = _tiles(jnp.stack([subj_edge_index[0], obj_edge_index[0]]))
    dst_t = _tiles(jnp.stack([subj_edge_index[1], obj_edge_index[1]]))

    a2o, a2i, cd, cs, _h1 = _sc_aggregate(po, pi, src_t, dst_t, b_O1, b_I1)

    n2g = jnp.stack([subj_node2graph, obj_node2graph])
    n2g = jnp.pad(n2g, ((0, 0), (0, NP - N)), constant_values=B)

    cd = cd.reshape(2 * NB, 1, BLKC)
    cs = cs.reshape(2 * NB, 1, BLKC)
    n2g = n2g.reshape(2 * NB, 1, BLKC)

    pos, neg = _finish(a2o, a2i, cd, cs, n2g, W_O2, b_O2, W_I2, b_I2,
                       rel_tok_embs, W_rel, b_rel, head_or_tail, random_rows)
    return (pos, neg)


# speedup vs baseline: 1.1649x; 1.0117x over previous
import functools

import jax
import jax.numpy as jnp
from jax import lax
from jax.experimental import pallas as pl
from jax.experimental.pallas import tpu as pltpu
from jax.experimental.pallas import tpu_sc as plsc

N = 10000
E = 320000
B = 1024
D = 128
DH = 32

NS = 16
NP = 10240
RPT = NP // NS
EPT = E // NS
CH = 128
NCH = 160
EPP = NCH * CH


def _front_body(xs_ref, xo_ref, wo_ref, wi_ref, po_ref, pi_ref):
    wo = wo_ref[...]
    wi = wi_ref[...]
    zpad = jnp.zeros((NP - N, DH), jnp.float32)
    for g, xr in ((0, xs_ref), (1, xo_ref)):
        x = xr[...]
        po_ref[g, :N, :] = jnp.dot(x, wo, preferred_element_type=jnp.float32)
        po_ref[g, N:, :] = zpad
        pi_ref[g, :N, :] = jnp.dot(x, wi, preferred_element_type=jnp.float32)
        pi_ref[g, N:, :] = zpad


def _front(xs, xo, wo, wi):
    return pl.pallas_call(
        _front_body,
        out_shape=[
            jax.ShapeDtypeStruct((2, NP, DH), jnp.float32),
            jax.ShapeDtypeStruct((2, NP, DH), jnp.float32),
        ],
    )(xs, xo, wo, wi)


CR = 160
NCR = RPT // CR
SCH = 512
NSC = NCH * CH // SCH
BLKQ = 8


def _sc_body(po_hbm, pi_hbm, src_hbm, dst_hbm, bo_hbm, bi_hbm,
             a2o_hbm, a2i_hbm, cd_hbm, cs_hbm, h1_hbm,
             acc_o, acc_i, cd_sh, cs_sh,
             sidx, didx, gb_a, gb_b,
             st_a, st_b, st_h,
             cd_l, cs_l, ivd, ivs, ones_v, bo_v, bi_v,
             sem_ga, sem_gb, sem_sa, sem_sb, sem_c):
    c = lax.axis_index("c")
    s = lax.axis_index("s")
    row0 = s * RPT
    rows = pl.ds(row0, RPT)
    f32 = jnp.float32

    def _chunk(k):
        return pl.ds(row0 + k * CR, CR)

    def _zero_rows(r, _):
        st_h[r, pl.ds(0, 16)] = jnp.zeros((16,), f32)
        st_h[r, pl.ds(16, 16)] = jnp.zeros((16,), f32)
        return 0

    lax.fori_loop(0, CR, _zero_rows, 0)

    def _zero_acc(k, _):
        pltpu.sync_copy(st_h, acc_o.at[_chunk(k), :])
        pltpu.sync_copy(st_h, acc_i.at[_chunk(k), :])
        return 0

    lax.fori_loop(0, NCR, _zero_acc, 0)

    def _zero_vec(k, _):
        ivd[pl.ds(k * 16, 16)] = jnp.zeros((16,), f32)
        return 0

    lax.fori_loop(0, RPT // 16, _zero_vec, 0)
    pltpu.sync_copy(ivd, cd_sh.at[rows])
    pltpu.sync_copy(ivd, cs_sh.at[rows])

    def _fill_ones(k, _):
        ones_v[pl.ds(k * 16, 16)] = jnp.zeros((16,), f32) + 1.0
        return 0

    lax.fori_loop(0, SCH // 16, _fill_ones, 0)
    pltpu.sync_copy(bo_hbm, bo_v)
    pltpu.sync_copy(bi_hbm, bi_v)

    plsc.subcore_barrier()

    def _dir(tab, gather_src, acc, with_counts):
        def blk_body(b, _):
            jsl = pl.ds(b * BLKQ * SCH, BLKQ * SCH)
            pltpu.sync_copy(src_hbm.at[c, s, jsl], sidx)
            pltpu.sync_copy(dst_hbm.at[c, s, jsl], didx)
            gref, sref = (sidx, didx) if gather_src else (didx, sidx)

            def _g(q):
                return gref.at[pl.ds(q * SCH, SCH)]

            def _s(q):
                return sref.at[pl.ds(q * SCH, SCH)]

            nsc = BLKQ
            pltpu.async_copy(tab.at[_g(0)], gb_a, sem_ga)

            def pair(jj, _p):
                q0 = 2 * jj
                q1 = q0 + 1
                if with_counts:
                    pltpu.async_copy(
                        ones_v, cd_sh.at[didx.at[pl.ds(q0 * SCH, SCH)]],
                        sem_c, add=True)
                    pltpu.async_copy(
                        ones_v, cs_sh.at[sidx.at[pl.ds(q0 * SCH, SCH)]],
                        sem_c, add=True)
                    pltpu.async_copy(
                        ones_v, cd_sh.at[didx.at[pl.ds(q1 * SCH, SCH)]],
                        sem_c, add=True)
                    pltpu.async_copy(
                        ones_v, cs_sh.at[sidx.at[pl.ds(q1 * SCH, SCH)]],
                        sem_c, add=True)
                pltpu.make_async_copy(tab.at[_g(q0)], gb_a, sem_ga).wait()
                d_gb = pltpu.async_copy(tab.at[_g(q1)], gb_b, sem_gb)
                d_sa = pltpu.async_copy(gb_a, acc.at[_s(q0)], sem_sa,
                                        add=True)
                d_gb.wait()
                d_sa.wait()

                @pl.when(jj < nsc // 2 - 1)
                def _next():
                    pltpu.async_copy(tab.at[_g(q0 + 2)], gb_a, sem_ga)

                d_sb = pltpu.async_copy(gb_b, acc.at[_s(q1)], sem_sb,
                                        add=True)
                d_sb.wait()
                if with_counts:
                    for qx in (q0, q1):
                        pltpu.make_async_copy(
                            ones_v, cd_sh.at[didx.at[pl.ds(qx * SCH, SCH)]],
                            sem_c).wait()
                        pltpu.make_async_copy(
                            ones_v, cs_sh.at[sidx.at[pl.ds(qx * SCH, SCH)]],
                            sem_c).wait()
                return 0

            lax.fori_loop(0, nsc // 2, pair, 0)
            return 0

        lax.fori_loop(0, NSC // BLKQ, blk_body, 0)

    _dir(po_hbm.at[c], True, acc_o, True)
    _dir(pi_hbm.at[c], False, acc_i, False)
    plsc.subcore_barrier()

    pltpu.sync_copy(cd_sh.at[rows], cd_l)
    pltpu.sync_copy(cs_sh.at[rows], cs_l)

    def _inv(k, _):
        sl = pl.ds(k * 16, 16)
        one = jnp.zeros((16,), f32) + 1.0
        ivd[sl] = one / jnp.maximum(cd_l[sl], 1.0)
        ivs[sl] = one / jnp.maximum(cs_l[sl], 1.0)
        return 0

    lax.fori_loop(0, RPT // 16, _inv, 0)

    b1lo = bo_v[pl.ds(0, 16)] + bi_v[pl.ds(0, 16)]
    b1hi = bo_v[pl.ds(16, 16)] + bi_v[pl.ds(16, 16)]

    def _h1_chunk(k, _):
        pltpu.sync_copy(acc_o.at[_chunk(k), :], st_a)
        pltpu.sync_copy(acc_i.at[_chunk(k), :], st_b)

        def _h1_row(r, _r):
            i16 = jnp.zeros((16,), jnp.int32) + (k * CR + r)
            vd = plsc.load_gather(ivd, [i16])
            vs = plsc.load_gather(ivs, [i16])
            lo = st_a[r, pl.ds(0, 16)] * vd + st_b[r, pl.ds(0, 16)] * vs + b1lo
            hi = st_a[r, pl.ds(16, 16)] * vd + st_b[r, pl.ds(16, 16)] * vs + b1hi
            st_h[r, pl.ds(0, 16)] = jnp.maximum(lo, 0.0)
            st_h[r, pl.ds(16, 16)] = jnp.maximum(hi, 0.0)
            return 0

        lax.fori_loop(0, CR, _h1_row, 0)
        pltpu.sync_copy(st_h, h1_hbm.at[c, _chunk(k), :])
        return 0

    lax.fori_loop(0, NCR, _h1_chunk, 0)

    def _zero_rows2(r, _):
        st_a[r, pl.ds(0, 16)] = jnp.zeros((16,), f32)
        st_a[r, pl.ds(16, 16)] = jnp.zeros((16,), f32)
        return 0

    lax.fori_loop(0, CR, _zero_rows2, 0)

    def _zero_acc2(k, _):
        pltpu.sync_copy(st_a, acc_o.at[_chunk(k), :])
        pltpu.sync_copy(st_a, acc_i.at[_chunk(k), :])
        return 0

    lax.fori_loop(0, NCR, _zero_acc2, 0)
    plsc.subcore_barrier()

    _dir(h1_hbm.at[c], True, acc_o, False)
    _dir(h1_hbm.at[c], False, acc_i, False)
    plsc.subcore_barrier()

    def _out(k, _):
        pltpu.sync_copy(acc_o.at[_chunk(k), :], st_a)
        pltpu.sync_copy(st_a, a2o_hbm.at[c, _chunk(k), :])
        pltpu.sync_copy(acc_i.at[_chunk(k), :], st_b)
        pltpu.sync_copy(st_b, a2i_hbm.at[c, _chunk(k), :])
        return 0

    lax.fori_loop(0, NCR, _out, 0)
    pltpu.sync_copy(cd_l, cd_hbm.at[c, rows])
    pltpu.sync_copy(cs_l, cs_hbm.at[c, rows])


def _sc_aggregate(po, pi, src_t, dst_t, b_o1, b_i1):
    f32 = jnp.float32
    kern = pl.kernel(
        _sc_body,
        out_type=[
            jax.ShapeDtypeStruct((2, NP, DH), f32),
            jax.ShapeDtypeStruct((2, NP, DH), f32),
            jax.ShapeDtypeStruct((2, NP), f32),
            jax.ShapeDtypeStruct((2, NP), f32),
            jax.ShapeDtypeStruct((2, NP, DH), f32),
        ],
        mesh=plsc.VectorSubcoreMesh(core_axis_name="c", subcore_axis_name="s"),
        compiler_params=pltpu.CompilerParams(
            needs_layout_passes=False, use_tc_tiling_on_sc=False),
        scratch_types=[
            pltpu.VMEM_SHARED((NP, DH), f32),
            pltpu.VMEM_SHARED((NP, DH), f32),
            pltpu.VMEM_SHARED((NP,), f32),
            pltpu.VMEM_SHARED((NP,), f32),
            pltpu.VMEM((BLKQ * SCH,), jnp.int32),
            pltpu.VMEM((BLKQ * SCH,), jnp.int32),
            pltpu.VMEM((SCH, DH), f32),
            pltpu.VMEM((SCH, DH), f32),
            pltpu.VMEM((CR, DH), f32),
            pltpu.VMEM((CR, DH), f32),
            pltpu.VMEM((CR, DH), f32),
            pltpu.VMEM((RPT,), f32),
            pltpu.VMEM((RPT,), f32),
            pltpu.VMEM((RPT,), f32),
            pltpu.VMEM((RPT,), f32),
            pltpu.VMEM((SCH,), f32),
            pltpu.VMEM((DH,), f32),
            pltpu.VMEM((DH,), f32),
        ] + [pltpu.SemaphoreType.DMA] * 5 + [
        ],
    )
    return kern(po, pi, src_t, dst_t, b_o1, b_i1)


BLKC = 2048
NB = NP // BLKC


def _finish_body(a2o_ref, a2i_ref, cd_ref, cs_ref, n2g_ref,
                 wo2_ref, bo2_ref, wi2_ref, bi2_ref,
                 rel_ref, wr_ref, br_ref, hot_ref, rr_ref,
                 pos_ref, neg_ref, gsum, gcnt, g0s):
    g = pl.program_id(0)
    nb = pl.program_id(1)

    @pl.when(nb == 0)
    def _init():
        gsum[...] = jnp.zeros_like(gsum)
        gcnt[...] = jnp.zeros_like(gcnt)

    cd = cd_ref[0, 0, :]
    cs = cs_ref[0, 0, :]
    mo = a2o_ref[0] * (1.0 / jnp.maximum(cd, 1.0))[:, None]
    mi = a2i_ref[0] * (1.0 / jnp.maximum(cs, 1.0))[:, None]
    h2 = jnp.dot(mo, wo2_ref[...], preferred_element_type=jnp.float32)
    h2 = h2 + jnp.dot(mi, wi2_ref[...], preferred_element_type=jnp.float32)
    h2 = jnp.maximum(h2 + bo2_ref[...] + bi2_ref[...], 0.0)

    n2 = n2g_ref[0, 0, :]
    oh = (n2[:, None] == lax.broadcasted_iota(jnp.int32, (BLKC, B), 1))
    ohb = oh.astype(jnp.bfloat16)
    gsum[...] += lax.dot_general(
        ohb, h2.astype(jnp.bfloat16), (((0,), (0,)), ((), ())),
        preferred_element_type=jnp.float32)
    gcnt[...] += jnp.sum(oh.astype(jnp.float32), axis=0)

    @pl.when((g == 0) & (nb == NB - 1))
    def _stash():
        g0s[...] = gsum[...] * (1.0 / jnp.maximum(gcnt[...], 1.0))[:, None]

    @pl.when((g == 1) & (nb == NB - 1))
    def _score():
        g1 = gsum[...] * (1.0 / jnp.maximum(gcnt[...], 1.0))[:, None]
        g0 = g0s[...]
        r = jnp.dot(rel_ref[...], wr_ref[...],
                    preferred_element_type=jnp.float32)
        r = jnp.maximum(r + br_ref[...], 0.0)
        ar = lax.broadcasted_iota(jnp.int32, (B, 1), 0)[:, 0]
        hot = hot_ref[...]
        rr = rr_ref[...]
        chidx = jnp.where(hot == 1, rr, ar)
        ctidx = jnp.where(hot == 0, rr, ar)
        iota_b = lax.broadcasted_iota(jnp.int32, (B, B), 1)
        hsel = (chidx[:, None] == iota_b).astype(jnp.bfloat16)
        tsel = (ctidx[:, None] == iota_b).astype(jnp.bfloat16)
        gh = jnp.dot(hsel, g0.astype(jnp.bfloat16),
                     preferred_element_type=jnp.float32)
        gt = jnp.dot(tsel, g1.astype(jnp.bfloat16),
                     preferred_element_type=jnp.float32)
        dpos = g0 + r - g1
        dneg = gh + r - gt
        pos_ref[...] = jnp.sqrt(jnp.sum(dpos * dpos, axis=1))
        neg_ref[...] = jnp.sqrt(jnp.sum(dneg * dneg, axis=1))


def _finish(a2o, a2i, cd, cs, n2g, wo2, bo2, wi2, bi2,
            rel, wr, br, hot, rr):
    f32 = jnp.float32
    full = lambda shape: pl.BlockSpec(shape, lambda g, nb: tuple(0 for _ in shape))
    return pl.pallas_call(
        _finish_body,
        grid=(2, NB),
        in_specs=[
            pl.BlockSpec((1, BLKC, DH), lambda g, nb: (g, nb, 0)),
            pl.BlockSpec((1, BLKC, DH), lambda g, nb: (g, nb, 0)),
            pl.BlockSpec((1, 1, BLKC), lambda g, nb: (g * NB + nb, 0, 0)),
            pl.BlockSpec((1, 1, BLKC), lambda g, nb: (g * NB + nb, 0, 0)),
            pl.BlockSpec((1, 1, BLKC), lambda g, nb: (g * NB + nb, 0, 0)),
            full((DH, D)),
            full((D,)),
            full((DH, D)),
            full((D,)),
            full((B, D)),
            full((D, D)),
            full((D,)),
            full((B,)),
            full((B,)),
        ],
        out_specs=[
            pl.BlockSpec((B,), lambda g, nb: (0,)),
            pl.BlockSpec((B,), lambda g, nb: (0,)),
        ],
        out_shape=[
            jax.ShapeDtypeStruct((B,), f32),
            jax.ShapeDtypeStruct((B,), f32),
        ],
        scratch_shapes=[
            pltpu.VMEM((B, D), f32),
            pltpu.VMEM((B,), f32),
            pltpu.VMEM((B, D), f32),
        ],
    )(a2o, a2i, cd, cs, n2g, wo2, bo2, wi2, bi2, rel, wr, br, hot, rr)


def kernel(subj_node_embs, rel_tok_embs, obj_node_embs, subj_edge_index,
           obj_edge_index, subj_node2graph, obj_node2graph, head_or_tail,
           random_rows, W_O1, b_O1, W_I1, b_I1, W_O2, b_O2, W_I2, b_I2,
           W_rel, b_rel):
    po, pi = _front(subj_node_embs, obj_node_embs, W_O1, W_I1)

    pad = N + (jnp.arange(EPP - EPT, dtype=jnp.int32) % (NP - N))
    pad = jnp.broadcast_to(pad, (2, NS, EPP - EPT))

    def _tiles(row):
        r = row.reshape(2, NS, EPT)
        return jnp.concatenate([r, pad], axis=2).reshape(2, NS, NSC * SCH)

    src_t = _tiles(jnp.stack([subj_edge_index[0], obj_edge_index[0]]))
    dst_t = _tiles(jnp.stack([subj_edge_index[1], obj_edge_index[1]]))

    a2o, a2i, cd, cs, _h1 = _sc_aggregate(po, pi, src_t, dst_t, b_O1, b_I1)

    n2g = jnp.stack([subj_node2graph, obj_node2graph])
    n2g = jnp.pad(n2g, ((0, 0), (0, NP - N)), constant_values=B)

    cd = cd.reshape(2 * NB, 1, BLKC)
    cs = cs.reshape(2 * NB, 1, BLKC)
    n2g = n2g.reshape(2 * NB, 1, BLKC)

    pos, neg = _finish(a2o, a2i, cd, cs, n2g, W_O2, b_O2, W_I2, b_I2,
                       rel_tok_embs, W_rel, b_rel, head_or_tail, random_rows)
    return (pos, neg)
